# baseline (speedup 1.0000x reference)
import jax
import jax.numpy as jnp
from jax.experimental import pallas as pl
from jax.experimental.pallas import tpu as pltpu

N = 10000
BM = 400
RB = N // BM
DH = 32


def _gcn_kernel(x_ref, adj_ref, w0_ref, b0_ref, w1_ref, b1_ref, w2_ref, b2_ref,
                out_ref, ga_ref, gb_ref):
    p = pl.program_id(0)
    i = pl.program_id(1)

    @pl.when(jnp.logical_and(p == 0, i == 0))
    def _prolog():
        ga_ref[...] = jnp.dot(x_ref[...], w0_ref[...],
                              preferred_element_type=jnp.float32)

    @pl.when(p == 0)
    def _pass0():
        t = jnp.dot(adj_ref[...], ga_ref[...],
                    preferred_element_type=jnp.float32)
        h = jnp.maximum(t + b0_ref[...], 0.0)
        gb_ref[pl.ds(i * BM, BM), :] = jnp.dot(
            h, w1_ref[...], preferred_element_type=jnp.float32)

    @pl.when(p == 1)
    def _pass1():
        t = jnp.dot(adj_ref[...], gb_ref[...],
                    preferred_element_type=jnp.float32)
        h = jnp.maximum(t + b1_ref[...], 0.0)
        ga_ref[pl.ds(i * BM, BM), 0:1] = jnp.dot(
            h, w2_ref[...], preferred_element_type=jnp.float32)

    @pl.when(p == 2)
    def _pass2():
        t = jnp.dot(adj_ref[...], ga_ref[:, 0:1],
                    preferred_element_type=jnp.float32)
        out_ref[...] = jnp.maximum(t + b2_ref[...], 0.0)


def kernel(x, adj, W0, b0, W1, b1, W2, b2):
    out = pl.pallas_call(
        _gcn_kernel,
        grid=(3, RB),
        in_specs=[
            pl.BlockSpec((N, 128), lambda p, i: (0, 0)),
            pl.BlockSpec((BM, N), lambda p, i: (i, 0)),
            pl.BlockSpec((128, DH), lambda p, i: (0, 0)),
            pl.BlockSpec((1, DH), lambda p, i: (0, 0)),
            pl.BlockSpec((DH, DH), lambda p, i: (0, 0)),
            pl.BlockSpec((1, DH), lambda p, i: (0, 0)),
            pl.BlockSpec((DH, 1), lambda p, i: (0, 0)),
            pl.BlockSpec((1, 1), lambda p, i: (0, 0)),
        ],
        out_specs=pl.BlockSpec((BM, 1), lambda p, i: (i, 0)),
        out_shape=jax.ShapeDtypeStruct((N, 1), jnp.float32),
        scratch_shapes=[
            pltpu.VMEM((N, DH), jnp.float32),
            pltpu.VMEM((N, DH), jnp.float32),
        ],
    )(x, adj, W0, b0.reshape(1, DH), W1, b1.reshape(1, DH),
      W2, b2.reshape(1, 1))
    return out.reshape(N)
